# baseline (device time: 546832 ns/iter reference)
import jax

jax.config.update("jax_compilation_cache_dir", "/tmp/jax_cache")
jax.config.update("jax_persistent_cache_min_entry_size_bytes", -1)
jax.config.update("jax_persistent_cache_min_compile_time_secs", 0)

import jax.numpy as jnp
from jax import lax
from jax.experimental import pallas as pl
from jax.experimental.pallas import tpu as pltpu

B = 4
S = 1024
S_HALF = 512
R = 128
K = 2048
N = 4096
N_ROUTE = 8
N_CHUNK = 16


def kernel(O, Wo):
    O2 = O.reshape(B, S, K)

    def body(o_ref, wo_ref, out_ref, ystage_ref, xstage_ref,
             o_buf, s_buf, add_ld,
             o_sems, ysend_sems, yrecv_sems, relay_sems, xrecv_sems,
             ld_sems, store_sems):
        my_x = lax.axis_index("x")
        my_y = lax.axis_index("y")
        peer_y = (my_x, 1 - my_y)
        peer_x = (1 - my_x, my_y)
        my_lo = my_y * S_HALF
        peer_lo = (1 - my_y) * S_HALF

        def oA_desc(a):
            b = a // 2
            r = (a % 2) * 2 + my_x
            return pltpu.make_async_copy(
                o_ref.at[b, pl.ds(peer_lo + r * R, R), :],
                o_buf.at[a % 2], o_sems.at[a % 2])

        def oC_desc(c):
            b, r = c // 4, c % 4
            return pltpu.make_async_copy(
                o_ref.at[b, pl.ds(my_lo + r * R, R), :],
                o_buf.at[c % 2], o_sems.at[c % 2])

        def ysend_desc(a):
            return pltpu.make_async_remote_copy(
                src_ref=s_buf.at[a], dst_ref=ystage_ref.at[a],
                send_sem=ysend_sems.at[a], recv_sem=yrecv_sems.at[a],
                device_id=peer_y, device_id_type=pl.DeviceIdType.MESH)

        def relay_desc(a):
            return pltpu.make_async_remote_copy(
                src_ref=ystage_ref.at[a], dst_ref=xstage_ref.at[a],
                send_sem=relay_sems.at[a], recv_sem=xrecv_sems.at[a],
                device_id=peer_x, device_id_type=pl.DeviceIdType.MESH)

        def ld_desc(stage_ref, a, slot):
            return pltpu.make_async_copy(
                stage_ref.at[a], add_ld.at[slot], ld_sems.at[slot])

        def store_desc(c, slot):
            b, r = c // 4, c % 4
            return pltpu.make_async_copy(
                add_ld.at[slot],
                out_ref.at[b, pl.ds(r * R, R), :],
                store_sems.at[slot])

        oA_desc(0).start()

        barrier = pltpu.get_barrier_semaphore()
        for nbr in (peer_y, peer_x):
            pl.semaphore_signal(barrier, inc=1, device_id=nbr,
                                device_id_type=pl.DeviceIdType.MESH)
        pl.semaphore_wait(barrier, 2)

        def phaseA(a, carry):
            oA_desc(a).wait()

            @pl.when(a < N_ROUTE - 1)
            def _():
                oA_desc(a + 1).start()

            @pl.when(a == N_ROUTE - 1)
            def _():
                oC_desc(0).start()

            s_buf[a] = jnp.dot(
                o_buf[a % 2], wo_ref[...], preferred_element_type=jnp.float32)
            ysend_desc(a).start()
            return carry

        lax.fori_loop(0, N_ROUTE, phaseA, 0)

        def phaseC(c, carry):
            r = c % 4
            a = (c // 4) * 2 + r // 2
            slot = c % 2
            is_y = (r % 2) == my_x
            oC_desc(c).wait()

            @pl.when(c < N_CHUNK - 1)
            def _():
                oC_desc(c + 1).start()

            @pl.when(c >= 2)
            def _():
                store_desc(c - 2, slot).wait()

            @pl.when(is_y)
            def _():
                ysend_desc(a).wait_recv()
                relay_desc(a).start()
                ld_desc(ystage_ref, a, slot).start()

            @pl.when(jnp.logical_not(is_y))
            def _():
                relay_desc(a).wait_recv()
                ld_desc(xstage_ref, a, slot).start()

            own = jnp.dot(
                o_buf[slot], wo_ref[...], preferred_element_type=jnp.float32)
            ld_desc(ystage_ref, a, slot).wait()
            add_ld[slot] = add_ld[slot] + own
            store_desc(c, slot).start()
            return carry

        lax.fori_loop(0, N_CHUNK, phaseC, 0)

        def drain(a, carry):
            ysend_desc(a).wait_send()
            relay_desc(a).wait_send()
            return carry

        lax.fori_loop(0, N_ROUTE, drain, 0)
        store_desc(N_CHUNK - 2, 0).wait()
        store_desc(N_CHUNK - 1, 1).wait()

    out = pl.pallas_call(
        body,
        out_shape=[
            jax.ShapeDtypeStruct((B, S_HALF, N), jnp.float32),
            jax.ShapeDtypeStruct((N_ROUTE, R, N), jnp.float32),
            jax.ShapeDtypeStruct((N_ROUTE, R, N), jnp.float32),
        ],
        in_specs=[
            pl.BlockSpec(memory_space=pl.ANY),
            pl.BlockSpec(memory_space=pltpu.VMEM),
        ],
        out_specs=[
            pl.BlockSpec(memory_space=pl.ANY),
            pl.BlockSpec(memory_space=pl.ANY),
            pl.BlockSpec(memory_space=pl.ANY),
        ],
        scratch_shapes=[
            pltpu.VMEM((2, R, K), jnp.float32),
            pltpu.VMEM((N_ROUTE, R, N), jnp.float32),
            pltpu.VMEM((2, R, N), jnp.float32),
            pltpu.SemaphoreType.DMA((2,)),
            pltpu.SemaphoreType.DMA((N_ROUTE,)),
            pltpu.SemaphoreType.DMA((N_ROUTE,)),
            pltpu.SemaphoreType.DMA((N_ROUTE,)),
            pltpu.SemaphoreType.DMA((N_ROUTE,)),
            pltpu.SemaphoreType.DMA((2,)),
            pltpu.SemaphoreType.DMA((2,)),
        ],
        compiler_params=pltpu.CompilerParams(
            collective_id=0, vmem_limit_bytes=63 * 1024 * 1024,
            has_side_effects=True,
        ),
    )(O2, Wo)
    return out[0]


# device time: 408266 ns/iter; 1.3394x vs baseline; 1.3394x over previous
import jax

jax.config.update("jax_compilation_cache_dir", "/tmp/jax_cache")
jax.config.update("jax_persistent_cache_min_entry_size_bytes", -1)
jax.config.update("jax_persistent_cache_min_compile_time_secs", 0)

import jax.numpy as jnp
from jax import lax
from jax.experimental import pallas as pl
from jax.experimental.pallas import tpu as pltpu

B = 4
S = 1024
S_HALF = 512
R = 256
K = 2048
N = 4096
N_CHUNK = B * (S_HALF // R)
N_SEND = 3


def kernel(O, Wo):

    def body(o_ref, wo_ref, out_ref, o_buf, s_buf, a_buf,
             o_sems, ld_sems, store_sems, send_sems, recv_sems):
        my_x = lax.axis_index("x")
        my_y = lax.axis_index("y")
        peer = (my_x, 1 - my_y)
        my_lo = my_y * S_HALF
        peer_lo = (1 - my_y) * S_HALF

        def o_load_desc(half_lo, c):
            b, j = c // 2, c % 2
            return pltpu.make_async_copy(
                o_ref.at[b, pl.ds(half_lo + j * R, R), :, :],
                o_buf.at[c % 2],
                o_sems.at[c % 2],
            )

        def remote_desc(c):
            b, j = c // 2, c % 2
            return pltpu.make_async_remote_copy(
                src_ref=s_buf.at[c % N_SEND],
                dst_ref=out_ref.at[b, pl.ds(j * R, R), :],
                send_sem=send_sems.at[c % N_SEND],
                recv_sem=recv_sems.at[c],
                device_id=peer,
                device_id_type=pl.DeviceIdType.MESH,
            )

        def ld_desc(c):
            b, j = c // 2, c % 2
            return pltpu.make_async_copy(
                out_ref.at[b, pl.ds(j * R, R), :],
                a_buf.at[c % 2],
                ld_sems.at[c % 2],
            )

        def store_desc(c):
            b, j = c // 2, c % 2
            return pltpu.make_async_copy(
                a_buf.at[c % 2],
                out_ref.at[b, pl.ds(j * R, R), :],
                store_sems.at[c % 2],
            )

        o_load_desc(peer_lo, 0).start()

        barrier = pltpu.get_barrier_semaphore()
        pl.semaphore_signal(
            barrier, inc=1, device_id=peer, device_id_type=pl.DeviceIdType.MESH
        )
        pl.semaphore_wait(barrier, 1)

        def phase1(c, carry):
            o_load_desc(peer_lo, c).wait()

            @pl.when(c < N_CHUNK - 1)
            def _():
                o_load_desc(peer_lo, c + 1).start()

            @pl.when(c == N_CHUNK - 1)
            def _():
                o_load_desc(my_lo, 0).start()

            @pl.when(c >= N_SEND)
            def _():
                remote_desc(c - N_SEND).wait_send()

            s_buf[c % N_SEND] = jnp.dot(
                o_buf[c % 2].reshape(R, K), wo_ref[...],
                preferred_element_type=jnp.float32,
            )
            remote_desc(c).start()
            return carry

        lax.fori_loop(0, N_CHUNK, phase1, 0)

        def phase2(c, carry):
            o_load_desc(my_lo, c).wait()

            @pl.when(c < N_CHUNK - 1)
            def _():
                o_load_desc(my_lo, c + 1).start()

            @pl.when(c >= 2)
            def _():
                store_desc(c - 2).wait()

            part = jnp.dot(
                o_buf[c % 2].reshape(R, K), wo_ref[...],
                preferred_element_type=jnp.float32,
            )
            remote_desc(c).wait_recv()
            ld_desc(c).start()
            ld_desc(c).wait()
            a_buf[c % 2] = a_buf[c % 2] + part
            store_desc(c).start()
            return carry

        lax.fori_loop(0, N_CHUNK, phase2, 0)

        for c in range(N_CHUNK - N_SEND, N_CHUNK):
            remote_desc(c).wait_send()
        for c in range(N_CHUNK - 2, N_CHUNK):
            store_desc(c).wait()

    return pl.pallas_call(
        body,
        out_shape=jax.ShapeDtypeStruct((B, S_HALF, N), jnp.float32),
        in_specs=[
            pl.BlockSpec(memory_space=pl.ANY),
            pl.BlockSpec(memory_space=pltpu.VMEM),
        ],
        out_specs=pl.BlockSpec(memory_space=pl.ANY),
        scratch_shapes=[
            pltpu.VMEM((2, R, 16, 128), jnp.float32),
            pltpu.VMEM((N_SEND, R, N), jnp.float32),
            pltpu.VMEM((2, R, N), jnp.float32),
            pltpu.SemaphoreType.DMA((2,)),
            pltpu.SemaphoreType.DMA((2,)),
            pltpu.SemaphoreType.DMA((2,)),
            pltpu.SemaphoreType.DMA((N_SEND,)),
            pltpu.SemaphoreType.DMA((N_CHUNK,)),
        ],
        compiler_params=pltpu.CompilerParams(
            collective_id=0, vmem_limit_bytes=63 * 1024 * 1024
        ),
    )(O, Wo)


# device time: 261928 ns/iter; 2.0877x vs baseline; 1.5587x over previous
import jax

jax.config.update("jax_compilation_cache_dir", "/tmp/jax_cache")
jax.config.update("jax_persistent_cache_min_entry_size_bytes", -1)
jax.config.update("jax_persistent_cache_min_compile_time_secs", 0)

import jax.numpy as jnp
from jax import lax
from jax.experimental import pallas as pl
from jax.experimental.pallas import tpu as pltpu

B = 4
S = 1024
S_HALF = 512
R = 128
K = 2048
N = 4096
N_ROUTE = 8


def kernel(O, Wo):
    def body(o_ref, wo_ref, out_ref, ystage_ref, xstage_ref,
             o_buf, s_buf, add_ld,
             o_sems, ysend_sems, yrecv_sems, relay_sems, xrecv_sems,
             ld_sems, store_sems):
        my_x = lax.axis_index("x")
        my_y = lax.axis_index("y")
        peer_y = (my_x, 1 - my_y)
        peer_x = (1 - my_x, my_y)
        my_lo = my_y * S_HALF
        peer_lo = (1 - my_y) * S_HALF

        def o_desc(half_lo, a, parity):
            b = a // 2
            r = (a % 2) * 2 + parity
            return pltpu.make_async_copy(
                o_ref.at[b, pl.ds(half_lo + r * R, R), :, :],
                o_buf.at[a % 2], o_sems.at[a % 2])

        def ysend_desc(a):
            return pltpu.make_async_remote_copy(
                src_ref=s_buf.at[a], dst_ref=ystage_ref.at[a],
                send_sem=ysend_sems.at[a], recv_sem=yrecv_sems.at[a],
                device_id=peer_y, device_id_type=pl.DeviceIdType.MESH)

        def relay_desc(a):
            return pltpu.make_async_remote_copy(
                src_ref=ystage_ref.at[a], dst_ref=xstage_ref.at[a],
                send_sem=relay_sems.at[a], recv_sem=xrecv_sems.at[a],
                device_id=peer_x, device_id_type=pl.DeviceIdType.MESH)

        def ld_desc(stage_ref, a, slot):
            return pltpu.make_async_copy(
                stage_ref.at[a], add_ld.at[slot], ld_sems.at[slot])

        def store_desc(a, parity, slot):
            b = a // 2
            r = (a % 2) * 2 + parity
            return pltpu.make_async_copy(
                add_ld.at[slot],
                out_ref.at[b, pl.ds(r * R, R), :],
                store_sems.at[slot])

        oA = lambda a: o_desc(peer_lo, a, my_x)
        oC1 = lambda a: o_desc(my_lo, a, my_x)
        oC2 = lambda a: o_desc(my_lo, a, 1 - my_x)

        oA(0).start()

        barrier = pltpu.get_barrier_semaphore()
        for nbr in (peer_y, peer_x):
            pl.semaphore_signal(barrier, inc=1, device_id=nbr,
                                device_id_type=pl.DeviceIdType.MESH)
        pl.semaphore_wait(barrier, 2)

        def phaseA(a, carry):
            oA(a).wait()

            @pl.when(a < N_ROUTE - 1)
            def _():
                oA(a + 1).start()

            @pl.when(a == N_ROUTE - 1)
            def _():
                oC1(0).start()

            s_buf[a] = jnp.dot(
                o_buf[a % 2].reshape(R, K), wo_ref[...],
                preferred_element_type=jnp.float32)
            ysend_desc(a).start()
            return carry

        lax.fori_loop(0, N_ROUTE, phaseA, 0)

        def phaseC1(a, carry):
            slot = a % 2
            oC1(a).wait()

            @pl.when(a < N_ROUTE - 1)
            def _():
                oC1(a + 1).start()

            @pl.when(a == N_ROUTE - 1)
            def _():
                oC2(0).start()

            ysend_desc(a).wait_recv()
            relay_desc(a).start()

            @pl.when(a >= 2)
            def _():
                store_desc(a - 2, my_x, slot).wait()

            ld_desc(ystage_ref, a, slot).start()
            own = jnp.dot(
                o_buf[slot].reshape(R, K), wo_ref[...],
                preferred_element_type=jnp.float32)
            ld_desc(ystage_ref, a, slot).wait()
            add_ld[slot] = add_ld[slot] + own
            store_desc(a, my_x, slot).start()
            return carry

        lax.fori_loop(0, N_ROUTE, phaseC1, 0)

        def phaseC2(a, carry):
            slot = a % 2
            oC2(a).wait()

            @pl.when(a < N_ROUTE - 1)
            def _():
                oC2(a + 1).start()

            relay_desc(a).wait_recv()
            store_desc(a, my_x, slot).wait()
            ld_desc(xstage_ref, a, slot).start()
            own = jnp.dot(
                o_buf[slot].reshape(R, K), wo_ref[...],
                preferred_element_type=jnp.float32)
            ld_desc(xstage_ref, a, slot).wait()
            add_ld[slot] = add_ld[slot] + own
            store_desc(a, 1 - my_x, slot).start()
            return carry

        lax.fori_loop(0, N_ROUTE, phaseC2, 0)

        def drain(a, carry):
            ysend_desc(a).wait_send()
            relay_desc(a).wait_send()
            return carry

        lax.fori_loop(0, N_ROUTE, drain, 0)
        store_desc(N_ROUTE - 2, 1 - my_x, 0).wait()
        store_desc(N_ROUTE - 1, 1 - my_x, 1).wait()

    out = pl.pallas_call(
        body,
        out_shape=[
            jax.ShapeDtypeStruct((B, S_HALF, N), jnp.float32),
            jax.ShapeDtypeStruct((N_ROUTE, R, N), jnp.float32),
            jax.ShapeDtypeStruct((N_ROUTE, R, N), jnp.float32),
        ],
        in_specs=[
            pl.BlockSpec(memory_space=pl.ANY),
            pl.BlockSpec(memory_space=pltpu.VMEM),
        ],
        out_specs=[
            pl.BlockSpec(memory_space=pl.ANY),
            pl.BlockSpec(memory_space=pl.ANY),
            pl.BlockSpec(memory_space=pl.ANY),
        ],
        scratch_shapes=[
            pltpu.VMEM((2, R, 16, 128), jnp.float32),
            pltpu.VMEM((N_ROUTE, R, N), jnp.float32),
            pltpu.VMEM((2, R, N), jnp.float32),
            pltpu.SemaphoreType.DMA((2,)),
            pltpu.SemaphoreType.DMA((N_ROUTE,)),
            pltpu.SemaphoreType.DMA((N_ROUTE,)),
            pltpu.SemaphoreType.DMA((N_ROUTE,)),
            pltpu.SemaphoreType.DMA((N_ROUTE,)),
            pltpu.SemaphoreType.DMA((2,)),
            pltpu.SemaphoreType.DMA((2,)),
        ],
        compiler_params=pltpu.CompilerParams(
            collective_id=0, vmem_limit_bytes=63 * 1024 * 1024,
            has_side_effects=True,
        ),
    )(O, Wo)
    return out[0]
